# baseline (speedup 1.0000x reference)
import jax
import jax.numpy as jnp
from jax import lax
from jax.experimental import pallas as pl
from jax.experimental.pallas import tpu as pltpu
from jax.experimental.pallas import tpu_sc as plsc

MS = 512
NT = MS * (MS + 1) // 2
BATCH = 128
CHUNKS = 16
RPC = MS // CHUNKS
LANES = 16
GPR = MS // LANES
BUF = 16384
OSZ = RPC * MS
HSZ = OSZ // 2
NBUF = 3
ZDELTA = 2 * NBUF


def _start(r: int) -> int:
    return r * (2 * MS - 1 - r) // 2


_CHUNK_LO = []
_CHUNK_SPAN = []
for _k in range(CHUNKS):
    _r0 = _k * RPC
    _lo = _start(_r0) & ~15
    _hi = (_start(_r0 + RPC - 1) + MS + 15) & ~15
    _CHUNK_LO.append(_lo)
    _CHUNK_SPAN.append(_hi - _lo)
assert max(_CHUNK_SPAN) <= BUF and _CHUNK_LO[-1] + _CHUNK_SPAN[-1] <= NT

_PREV_USE = [max(k for k in range(CHUNKS) if k % NBUF == s) for s in range(NBUF)]


def _sc_body(in_hbm, out_hbm, inbuf, outbuf, *sems):
    info = plsc.get_sparse_core_info()
    nc = info.num_cores
    wid = lax.axis_index("s") * nc + lax.axis_index("c")
    lane = lax.iota(jnp.int32, LANES)
    zvec = jnp.zeros((LANES,), jnp.float32)
    bpw = BATCH // 32
    sin = sems[:NBUF]
    sout = sems[NBUF:]

    def in_copy(b, k):
        cur = k % NBUF
        return pltpu.make_async_copy(
            in_hbm.at[pl.ds(b * NT + _CHUNK_LO[k], _CHUNK_SPAN[k])],
            inbuf.at[pl.ds(cur * BUF, _CHUNK_SPAN[k])],
            sin[cur])

    def out_copy(b, k, h):
        cur = k % NBUF
        return pltpu.make_async_copy(
            outbuf.at[pl.ds(cur * OSZ + h * HSZ, HSZ)],
            out_hbm.at[pl.ds((b * MS + k * RPC + 16 * h) * MS, HSZ)],
            sout[cur])

    def batch_body(bb, carry):
        b = wid * bpw + bb
        for k in range(NBUF - 1):
            in_copy(b, k).start()
        for k in range(CHUNKS):
            cur = k % NBUF
            r0 = k * RPC
            lo = _CHUNK_LO[k]
            if k + NBUF - 1 < CHUNKS:
                in_copy(b, k + NBUF - 1).start()
            in_copy(b, k).wait()
            if k < NBUF:
                @pl.when(bb > 0)
                def _():
                    out_copy(b, _PREV_USE[cur], 0).wait()
                    out_copy(b, _PREV_USE[cur], 1).wait()
            else:
                out_copy(b, k - NBUF, 0).wait()
                out_copy(b, k - NBUF, 1).wait()

            for h in range(2):
                g = 2 * k + h
                o0 = _start(r0 + 16 * h) - lo

                @plsc.parallel_loop(0, 16, carry=jnp.int32(o0))
                def row_body(t, o, g=g, h=h, r0=r0, cur=cur):
                    r = r0 + 16 * h + t
                    rb = cur * OSZ + (16 * h + t) * MS
                    for j in range(max(0, g - ZDELTA), g):
                        outbuf[pl.ds(rb + LANES * j, LANES)] = zvec
                    v = plsc.load_gather(
                        inbuf, [cur * BUF + o + LANES * g + lane])
                    outbuf[pl.ds(rb + LANES * g, LANES)] = (
                        jnp.where(lane >= t, v, 0.0))
                    for j in range(g + 1, GPR):
                        vj = plsc.load_gather(
                            inbuf, [cur * BUF + o + LANES * j + lane])
                        outbuf[pl.ds(rb + LANES * j, LANES)] = vj
                    return o + (MS - 1 - r)

                out_copy(b, k, h).start()
        return carry

    lax.fori_loop(0, bpw, batch_body, jnp.int32(0))
    for s in range(NBUF):
        out_copy(0, _PREV_USE[s], 0).wait()
        out_copy(0, _PREV_USE[s], 1).wait()


@jax.jit
def _triu_to_dense(inputs):
    mesh = plsc.VectorSubcoreMesh(core_axis_name="c", subcore_axis_name="s")
    flat = pl.kernel(
        _sc_body,
        mesh=mesh,
        compiler_params=pltpu.CompilerParams(needs_layout_passes=False),
        out_type=jax.ShapeDtypeStruct((BATCH * MS * MS,), jnp.float32),
        scratch_types=(
            [pltpu.VMEM((NBUF * BUF,), jnp.float32),
             pltpu.VMEM((NBUF * OSZ,), jnp.float32)]
            + [pltpu.SemaphoreType.DMA] * (2 * NBUF)
        ),
    )(inputs.reshape(-1))
    return flat.reshape(BATCH, MS, MS)


def kernel(inputs):
    return _triu_to_dense(inputs)
